# baseline (speedup 1.0000x reference)
import functools

import jax
import jax.numpy as jnp
from jax import lax
from jax.experimental import pallas as pl
from jax.experimental.pallas import tpu as pltpu
from jax.experimental.pallas import tpu_sc as plsc

N_NODES = 10000
NP = 10240
D = 128
E = 160000
NT = 16
EPT = E // NT
CHUNK = 100
NCHUNK = EPT // CHUNK
SUP = 10
SPC = NCHUNK // SUP
PAIRS = SPC // 2
RPT = NP // NT

_f32 = jnp.float32


def _sc_degrees_body(srcA, dstA, srcB, dstB, ones_h, z_h, degs,
                     acc, sidx, ones_v):
    c = lax.axis_index("c")
    t = lax.axis_index("s")
    rows = pl.ds(t * RPT, RPT)
    pltpu.sync_copy(z_h, acc.at[rows])
    pltpu.sync_copy(ones_h, ones_v)
    plsc.subcore_barrier()

    def accumulate(idxA, idxB):
        def step(j, carry):
            @pl.when(c == 0)
            def _():
                pltpu.sync_copy(idxA.at[t].at[j], sidx)

            @pl.when(c == 1)
            def _():
                pltpu.sync_copy(idxB.at[t].at[j], sidx)

            def inner(k, c2):
                pltpu.sync_copy(ones_v, acc.at[sidx.at[k]], add=True)
                return c2

            lax.fori_loop(0, SPC, inner, 0)
            return carry

        lax.fori_loop(0, SUP, step, 0)

    accumulate(srcA, srcB)
    plsc.subcore_barrier()

    @pl.when(c == 0)
    def _():
        pltpu.sync_copy(acc.at[rows], degs.at[0].at[rows])

    @pl.when(c == 1)
    def _():
        pltpu.sync_copy(acc.at[rows], degs.at[2].at[rows])

    plsc.subcore_barrier()
    pltpu.sync_copy(z_h, acc.at[rows])
    plsc.subcore_barrier()

    accumulate(dstA, dstB)
    plsc.subcore_barrier()

    @pl.when(c == 0)
    def _():
        pltpu.sync_copy(acc.at[rows], degs.at[1].at[rows])

    @pl.when(c == 1)
    def _():
        pltpu.sync_copy(acc.at[rows], degs.at[3].at[rows])


def _sc_degrees(srcA, dstA, srcB, dstB, ones_h, z_h):
    mesh = plsc.VectorSubcoreMesh(core_axis_name="c", subcore_axis_name="s")
    f = pl.kernel(
        _sc_degrees_body,
        out_type=jax.ShapeDtypeStruct((4, NP, D), _f32),
        mesh=mesh,
        scratch_types=[
            pltpu.VMEM_SHARED((NP, D), _f32),
            pltpu.VMEM((SPC, CHUNK), jnp.int32),
            pltpu.VMEM((CHUNK, D), _f32),
        ],
    )
    return f(srcA, dstA, srcB, dstB, ones_h, z_h)


def _sc_scatter_body(YA, YB, srcA, dstA, srcB, dstB, z_h, aggN, aggL,
                     acc, sidx, didx, buf0, buf1, sem0, sem1, semi, semd):
    c = lax.axis_index("c")
    t = lax.axis_index("s")
    pltpu.sync_copy(z_h, acc.at[pl.ds(t * RPT, RPT)])
    plsc.subcore_barrier()

    def run(Y, src4, dst4):
        s4 = src4.at[t]
        d4 = dst4.at[t]
        pltpu.sync_copy(s4.at[0], sidx.at[0])
        pltpu.sync_copy(d4.at[0], didx.at[0])
        pltpu.async_copy(Y.at[sidx.at[0].at[0]], buf0, sem0)

        def super_body(j, carry):
            cur = lax.rem(j, 2)
            nxt = lax.rem(j + 1, 2)

            @pl.when(j < SUP - 1)
            def _():
                pltpu.async_copy(s4.at[j + 1], sidx.at[nxt], semi)
                pltpu.async_copy(d4.at[j + 1], didx.at[nxt], semd)

            def pair(p, c2):
                k0 = 2 * p
                pltpu.make_async_copy(
                    Y.at[sidx.at[cur].at[k0]], buf0, sem0).wait()
                pltpu.async_copy(Y.at[sidx.at[cur].at[k0 + 1]], buf1, sem1)
                pltpu.sync_copy(buf0, acc.at[didx.at[cur].at[k0]], add=True)
                pltpu.make_async_copy(
                    Y.at[sidx.at[cur].at[k0 + 1]], buf1, sem1).wait()

                @pl.when(p < PAIRS - 1)
                def _():
                    pltpu.async_copy(Y.at[sidx.at[cur].at[k0 + 2]], buf0, sem0)

                @pl.when(p == PAIRS - 1)
                def _():
                    @pl.when(j < SUP - 1)
                    def _():
                        pltpu.make_async_copy(
                            s4.at[j + 1], sidx.at[nxt], semi).wait()
                        pltpu.make_async_copy(
                            d4.at[j + 1], didx.at[nxt], semd).wait()
                        pltpu.async_copy(Y.at[sidx.at[nxt].at[0]], buf0, sem0)

                pltpu.sync_copy(buf1, acc.at[didx.at[cur].at[k0 + 1]], add=True)
                return c2

            lax.fori_loop(0, PAIRS, pair, 0)
            return carry

        lax.fori_loop(0, SUP, super_body, 0)

    @pl.when(c == 0)
    def _():
        run(YA, srcA, dstA)

    @pl.when(c == 1)
    def _():
        run(YB, srcB, dstB)

    plsc.subcore_barrier()
    rows = pl.ds(t * RPT, RPT)

    @pl.when(c == 0)
    def _():
        pltpu.sync_copy(acc.at[rows], aggN.at[rows])

    @pl.when(c == 1)
    def _():
        pltpu.sync_copy(acc.at[rows], aggL.at[rows])


def _sc_scatter(YA, YB, srcA, dstA, srcB, dstB, z_h):
    mesh = plsc.VectorSubcoreMesh(core_axis_name="c", subcore_axis_name="s")
    f = pl.kernel(
        _sc_scatter_body,
        out_type=(
            jax.ShapeDtypeStruct((NP, D), _f32),
            jax.ShapeDtypeStruct((NP, D), _f32),
        ),
        mesh=mesh,
        scratch_types=[
            pltpu.VMEM_SHARED((NP, D), _f32),
            pltpu.VMEM((2, SPC, CHUNK), jnp.int32),
            pltpu.VMEM((2, SPC, CHUNK), jnp.int32),
            pltpu.VMEM((CHUNK, D), _f32),
            pltpu.VMEM((CHUNK, D), _f32),
            pltpu.SemaphoreType.DMA,
            pltpu.SemaphoreType.DMA,
            pltpu.SemaphoreType.DMA,
            pltpu.SemaphoreType.DMA,
        ],
    )
    return f(YA, YB, srcA, dstA, srcB, dstB, z_h)


def _sc_scatter_one_body(Y, src4, dst4, z_h, agg,
                         acc, sidx, didx, buf0, buf1, sem0, sem1, semi, semd):
    t = lax.axis_index("s")
    pltpu.sync_copy(z_h, acc.at[pl.ds(t * RPT, RPT)])
    plsc.subcore_barrier()

    s4 = src4.at[t]
    d4 = dst4.at[t]
    pltpu.sync_copy(s4.at[0], sidx.at[0])
    pltpu.sync_copy(d4.at[0], didx.at[0])
    pltpu.async_copy(Y.at[sidx.at[0].at[0]], buf0, sem0)

    def super_body(j, carry):
        cur = lax.rem(j, 2)
        nxt = lax.rem(j + 1, 2)

        @pl.when(j < SUP - 1)
        def _():
            pltpu.async_copy(s4.at[j + 1], sidx.at[nxt], semi)
            pltpu.async_copy(d4.at[j + 1], didx.at[nxt], semd)

        def pair(p, c2):
            k0 = 2 * p
            pltpu.make_async_copy(Y.at[sidx.at[cur].at[k0]], buf0, sem0).wait()
            pltpu.async_copy(Y.at[sidx.at[cur].at[k0 + 1]], buf1, sem1)
            pltpu.sync_copy(buf0, acc.at[didx.at[cur].at[k0]], add=True)
            pltpu.make_async_copy(Y.at[sidx.at[cur].at[k0 + 1]], buf1, sem1).wait()

            @pl.when(p < PAIRS - 1)
            def _():
                pltpu.async_copy(Y.at[sidx.at[cur].at[k0 + 2]], buf0, sem0)

            @pl.when(p == PAIRS - 1)
            def _():
                @pl.when(j < SUP - 1)
                def _():
                    pltpu.make_async_copy(s4.at[j + 1], sidx.at[nxt], semi).wait()
                    pltpu.make_async_copy(d4.at[j + 1], didx.at[nxt], semd).wait()
                    pltpu.async_copy(Y.at[sidx.at[nxt].at[0]], buf0, sem0)

            pltpu.sync_copy(buf1, acc.at[didx.at[cur].at[k0 + 1]], add=True)
            return c2

        lax.fori_loop(0, PAIRS, pair, 0)
        return carry

    lax.fori_loop(0, SUP, super_body, 0)
    plsc.subcore_barrier()
    rows = pl.ds(t * RPT, RPT)
    pltpu.sync_copy(acc.at[rows], agg.at[rows])


def _sc_scatter_one(Y, src4, dst4, z_h):
    mesh = plsc.VectorSubcoreMesh(core_axis_name="c", subcore_axis_name="s",
                                  num_cores=1)
    f = pl.kernel(
        _sc_scatter_one_body,
        out_type=jax.ShapeDtypeStruct((NP, D), _f32),
        mesh=mesh,
        scratch_types=[
            pltpu.VMEM_SHARED((NP, D), _f32),
            pltpu.VMEM((2, SPC, CHUNK), jnp.int32),
            pltpu.VMEM((2, SPC, CHUNK), jnp.int32),
            pltpu.VMEM((CHUNK, D), _f32),
            pltpu.VMEM((CHUNK, D), _f32),
            pltpu.SemaphoreType.DMA,
            pltpu.SemaphoreType.DMA,
            pltpu.SemaphoreType.DMA,
            pltpu.SemaphoreType.DMA,
        ],
    )
    return f(Y, src4, dst4, z_h)


_BLK = 1000


def _tc_scale_matmul(x, deg16, W):
    n, din = x.shape
    dout = W.shape[1]

    def body(x_ref, d_ref, w_ref, o_ref):
        s = lax.rsqrt(jnp.maximum(d_ref[:, 0:1], 1.0))
        o_ref[...] = jnp.dot(x_ref[...] * s, w_ref[...],
                             preferred_element_type=_f32)

    return pl.pallas_call(
        body,
        grid=(n // _BLK,),
        in_specs=[
            pl.BlockSpec((_BLK, din), lambda i: (i, 0)),
            pl.BlockSpec((_BLK, D), lambda i: (i, 0)),
            pl.BlockSpec((din, dout), lambda i: (0, 0)),
        ],
        out_specs=pl.BlockSpec((_BLK, dout), lambda i: (i, 0)),
        out_shape=jax.ShapeDtypeStruct((n, dout), _f32),
    )(x, deg16, W)


def _norm_elu(a_ref, di_ref, b_ref, g_ref, bb_ref):
    si = lax.rsqrt(jnp.maximum(di_ref[:, 0:1], 1.0))
    a = a_ref[...] * si + b_ref[...]
    mu = jnp.mean(a, axis=1, keepdims=True)
    var = jnp.mean((a - mu) ** 2, axis=1, keepdims=True)
    xh = (a - mu) * lax.rsqrt(var + 1e-5) * g_ref[...] + bb_ref[...]
    return jnp.where(xh > 0, xh, jnp.exp(jnp.minimum(xh, 0.0)) - 1.0)


def _tc_combine(agg, indeg16, outdeg16, b, g, bb, Wn):
    n = agg.shape[0]
    blk = 1024

    def body(a_ref, di_ref, do_ref, b_ref, g_ref, bb_ref, w_ref, o_ref):
        h = _norm_elu(a_ref, di_ref, b_ref, g_ref, bb_ref)
        so = lax.rsqrt(jnp.maximum(do_ref[:, 0:1], 1.0))
        o_ref[...] = jnp.dot(h * so, w_ref[...], preferred_element_type=_f32)

    return pl.pallas_call(
        body,
        grid=(n // blk,),
        in_specs=[
            pl.BlockSpec((blk, D), lambda i: (i, 0)),
            pl.BlockSpec((blk, D), lambda i: (i, 0)),
            pl.BlockSpec((blk, D), lambda i: (i, 0)),
            pl.BlockSpec((1, D), lambda i: (0, 0)),
            pl.BlockSpec((1, D), lambda i: (0, 0)),
            pl.BlockSpec((1, D), lambda i: (0, 0)),
            pl.BlockSpec((D, D), lambda i: (0, 0)),
        ],
        out_specs=pl.BlockSpec((blk, D), lambda i: (i, 0)),
        out_shape=jax.ShapeDtypeStruct((n, D), _f32),
    )(agg, indeg16, outdeg16, b, g, bb, Wn)


def _tc_final(aggN, aggL, indegN, indegL, bN, gN, bbN, bL, gL, bbL,
              Wfc, bfc, WoutT, bout):
    n = aggN.shape[0]
    blk = 1024
    steps = n // blk

    def body(an_ref, din_ref, bn_ref, gn_ref, bbn_ref,
             al_ref, dil_ref, bl_ref, gl_ref, bbl_ref,
             wfc_ref, bfc_ref, wout_ref, bout_ref, o_ref, acc_ref):
        i = pl.program_id(0)
        hn = _norm_elu(an_ref, din_ref, bn_ref, gn_ref, bbn_ref)
        hl = _norm_elu(al_ref, dil_ref, bl_ref, gl_ref, bbl_ref)
        row = lax.broadcasted_iota(jnp.int32, (blk, 1), 0) + i * blk
        valid = row < N_NODES
        zero = jnp.zeros_like(hn)
        part = (jnp.sum(jnp.where(valid, hn, zero), axis=0, keepdims=True)
                + jnp.sum(jnp.where(valid, hl, zero), axis=0, keepdims=True))

        @pl.when(i == 0)
        def _():
            acc_ref[0:1, :] = part

        @pl.when(i > 0)
        def _():
            acc_ref[0:1, :] = acc_ref[0:1, :] + part

        @pl.when(i == steps - 1)
        def _():
            hg = acc_ref[0:1, :] * _f32(1.0 / N_NODES)
            z = jnp.maximum(
                jnp.dot(hg, wfc_ref[...], preferred_element_type=_f32)
                + bfc_ref[...], 0.0)
            val = jnp.sum(z * wout_ref[...]) + bout_ref[0, 0]
            o_ref[...] = jnp.full((8, D), val, _f32)

    return pl.pallas_call(
        body,
        grid=(steps,),
        in_specs=[
            pl.BlockSpec((blk, D), lambda i: (i, 0)),
            pl.BlockSpec((blk, D), lambda i: (i, 0)),
            pl.BlockSpec((1, D), lambda i: (0, 0)),
            pl.BlockSpec((1, D), lambda i: (0, 0)),
            pl.BlockSpec((1, D), lambda i: (0, 0)),
            pl.BlockSpec((blk, D), lambda i: (i, 0)),
            pl.BlockSpec((blk, D), lambda i: (i, 0)),
            pl.BlockSpec((1, D), lambda i: (0, 0)),
            pl.BlockSpec((1, D), lambda i: (0, 0)),
            pl.BlockSpec((1, D), lambda i: (0, 0)),
            pl.BlockSpec((D, D), lambda i: (0, 0)),
            pl.BlockSpec((1, D), lambda i: (0, 0)),
            pl.BlockSpec((1, D), lambda i: (0, 0)),
            pl.BlockSpec((1, 1), lambda i: (0, 0)),
        ],
        out_specs=pl.BlockSpec((8, D), lambda i: (0, 0)),
        out_shape=jax.ShapeDtypeStruct((8, D), _f32),
        scratch_shapes=[pltpu.VMEM((8, D), _f32)],
    )(aggN, indegN, bN, gN, bbN, aggL, indegL, bL, gL, bbL,
      Wfc, bfc, WoutT, bout)


def kernel(feat_n, feat_l, edge_index_l2n, edge_index_n2l, params):
    srcA = edge_index_l2n[0].astype(jnp.int32).reshape(NT, SUP, SPC, CHUNK)
    dstA = edge_index_l2n[1].astype(jnp.int32).reshape(NT, SUP, SPC, CHUNK)
    srcB = edge_index_n2l[0].astype(jnp.int32).reshape(NT, SUP, SPC, CHUNK)
    dstB = edge_index_n2l[1].astype(jnp.int32).reshape(NT, SUP, SPC, CHUNK)

    z_h = jnp.zeros((RPT, D), _f32)
    ones_h = jnp.ones((CHUNK, D), _f32)

    degs = _sc_degrees(srcA, dstA, srcB, dstB, ones_h, z_h)
    outdegL, indegN, outdegN, indegL = degs[0], degs[1], degs[2], degs[3]

    def r2(v):
        return v.reshape(1, -1)

    Yl = _tc_scale_matmul(feat_l, outdegL, params['W_l2n'][0])
    Yn = _tc_scale_matmul(feat_n, outdegN, params['W_n2l'][0])

    for i in range(2):
        aggN = _sc_scatter_one(Yl, srcA, dstA, z_h)
        aggL = _sc_scatter_one(Yn, srcB, dstB, z_h)
        Yn = _tc_combine(aggN, indegN, outdegN, r2(params['b_l2n'][i]),
                         r2(params['ln_n_g'][i]), r2(params['ln_n_b'][i]),
                         params['W_n2l'][i + 1])
        Yl = _tc_combine(aggL, indegL, outdegL, r2(params['b_n2l'][i]),
                         r2(params['ln_l_g'][i]), r2(params['ln_l_b'][i]),
                         params['W_l2n'][i + 1])

    aggN = _sc_scatter_one(Yl, srcA, dstA, z_h)
    aggL = _sc_scatter_one(Yn, srcB, dstB, z_h)
    out8 = _tc_final(
        aggN, aggL, indegN, indegL,
        r2(params['b_l2n'][2]), r2(params['ln_n_g'][2]), r2(params['ln_n_b'][2]),
        r2(params['b_n2l'][2]), r2(params['ln_l_g'][2]), r2(params['ln_l_b'][2]),
        params['W_fc'], r2(params['b_fc']),
        params['W_out'].reshape(1, D), params['b_out'].reshape(1, 1))
    return out8[0:1, 0:1]


# speedup vs baseline: 1.4770x; 1.4770x over previous
import functools

import jax
import jax.numpy as jnp
from jax import lax
from jax.experimental import pallas as pl
from jax.experimental.pallas import tpu as pltpu
from jax.experimental.pallas import tpu_sc as plsc

N_NODES = 10000
NP = 10240
D = 128
E = 160000
NT = 16
EPT = E // NT
CHUNK = 100
NCHUNK = EPT // CHUNK
SUP = 10
SPC = NCHUNK // SUP
PAIRS = SPC // 2
RPT = NP // NT

_f32 = jnp.float32


def _sc_degrees_body(srcA, dstA, srcB, dstB, ones_h, z_h, degs,
                     acc, sidx, ones_v):
    c = lax.axis_index("c")
    t = lax.axis_index("s")
    rows = pl.ds(t * RPT, RPT)
    pltpu.sync_copy(z_h, acc.at[rows])
    pltpu.sync_copy(ones_h, ones_v)
    plsc.subcore_barrier()

    def accumulate(idxA, idxB):
        def step(j, carry):
            @pl.when(c == 0)
            def _():
                pltpu.sync_copy(idxA.at[t].at[j], sidx)

            @pl.when(c == 1)
            def _():
                pltpu.sync_copy(idxB.at[t].at[j], sidx)

            def inner(k, c2):
                pltpu.sync_copy(ones_v, acc.at[sidx.at[k]], add=True)
                return c2

            lax.fori_loop(0, SPC, inner, 0)
            return carry

        lax.fori_loop(0, SUP, step, 0)

    accumulate(srcA, srcB)
    plsc.subcore_barrier()

    @pl.when(c == 0)
    def _():
        pltpu.sync_copy(acc.at[rows], degs.at[0].at[rows])

    @pl.when(c == 1)
    def _():
        pltpu.sync_copy(acc.at[rows], degs.at[2].at[rows])

    plsc.subcore_barrier()
    pltpu.sync_copy(z_h, acc.at[rows])
    plsc.subcore_barrier()

    accumulate(dstA, dstB)
    plsc.subcore_barrier()

    @pl.when(c == 0)
    def _():
        pltpu.sync_copy(acc.at[rows], degs.at[1].at[rows])

    @pl.when(c == 1)
    def _():
        pltpu.sync_copy(acc.at[rows], degs.at[3].at[rows])


def _sc_degrees(srcA, dstA, srcB, dstB, ones_h, z_h):
    mesh = plsc.VectorSubcoreMesh(core_axis_name="c", subcore_axis_name="s")
    f = pl.kernel(
        _sc_degrees_body,
        out_type=jax.ShapeDtypeStruct((4, NP, D), _f32),
        mesh=mesh,
        scratch_types=[
            pltpu.VMEM_SHARED((NP, D), _f32),
            pltpu.VMEM((SPC, CHUNK), jnp.int32),
            pltpu.VMEM((CHUNK, D), _f32),
        ],
    )
    return f(srcA, dstA, srcB, dstB, ones_h, z_h)


def _sc_scatter_body(YA, YB, srcA, dstA, srcB, dstB, z_h, aggN, aggL,
                     acc, sidx, didx, buf0, buf1, sem0, sem1, semi, semd):
    c = lax.axis_index("c")
    t = lax.axis_index("s")
    pltpu.sync_copy(z_h, acc.at[pl.ds(t * RPT, RPT)])
    plsc.subcore_barrier()

    def run(Y, src4, dst4):
        s4 = src4.at[t]
        d4 = dst4.at[t]
        pltpu.sync_copy(s4.at[0], sidx.at[0])
        pltpu.sync_copy(d4.at[0], didx.at[0])
        pltpu.async_copy(Y.at[sidx.at[0].at[0]], buf0, sem0)

        def super_body(j, carry):
            cur = lax.rem(j, 2)
            nxt = lax.rem(j + 1, 2)

            @pl.when(j < SUP - 1)
            def _():
                pltpu.async_copy(s4.at[j + 1], sidx.at[nxt], semi)
                pltpu.async_copy(d4.at[j + 1], didx.at[nxt], semd)

            def pair(p, c2):
                k0 = 2 * p
                pltpu.make_async_copy(
                    Y.at[sidx.at[cur].at[k0]], buf0, sem0).wait()
                pltpu.async_copy(Y.at[sidx.at[cur].at[k0 + 1]], buf1, sem1)
                pltpu.sync_copy(buf0, acc.at[didx.at[cur].at[k0]], add=True)
                pltpu.make_async_copy(
                    Y.at[sidx.at[cur].at[k0 + 1]], buf1, sem1).wait()

                @pl.when(p < PAIRS - 1)
                def _():
                    pltpu.async_copy(Y.at[sidx.at[cur].at[k0 + 2]], buf0, sem0)

                @pl.when(p == PAIRS - 1)
                def _():
                    @pl.when(j < SUP - 1)
                    def _():
                        pltpu.make_async_copy(
                            s4.at[j + 1], sidx.at[nxt], semi).wait()
                        pltpu.make_async_copy(
                            d4.at[j + 1], didx.at[nxt], semd).wait()
                        pltpu.async_copy(Y.at[sidx.at[nxt].at[0]], buf0, sem0)

                pltpu.sync_copy(buf1, acc.at[didx.at[cur].at[k0 + 1]], add=True)
                return c2

            lax.fori_loop(0, PAIRS, pair, 0)
            return carry

        lax.fori_loop(0, SUP, super_body, 0)

    @pl.when(c == 0)
    def _():
        run(YA, srcA, dstA)

    @pl.when(c == 1)
    def _():
        run(YB, srcB, dstB)

    plsc.subcore_barrier()
    rows = pl.ds(t * RPT, RPT)

    @pl.when(c == 0)
    def _():
        pltpu.sync_copy(acc.at[rows], aggN.at[rows])

    @pl.when(c == 1)
    def _():
        pltpu.sync_copy(acc.at[rows], aggL.at[rows])


def _sc_scatter(YA, YB, srcA, dstA, srcB, dstB, z_h):
    mesh = plsc.VectorSubcoreMesh(core_axis_name="c", subcore_axis_name="s")
    f = pl.kernel(
        _sc_scatter_body,
        out_type=(
            jax.ShapeDtypeStruct((NP, D), _f32),
            jax.ShapeDtypeStruct((NP, D), _f32),
        ),
        mesh=mesh,
        scratch_types=[
            pltpu.VMEM_SHARED((NP, D), _f32),
            pltpu.VMEM((2, SPC, CHUNK), jnp.int32),
            pltpu.VMEM((2, SPC, CHUNK), jnp.int32),
            pltpu.VMEM((CHUNK, D), _f32),
            pltpu.VMEM((CHUNK, D), _f32),
            pltpu.SemaphoreType.DMA,
            pltpu.SemaphoreType.DMA,
            pltpu.SemaphoreType.DMA,
            pltpu.SemaphoreType.DMA,
        ],
    )
    return f(YA, YB, srcA, dstA, srcB, dstB, z_h)


_BLK = 1000


def _tc_matmul(x, W):
    n, din = x.shape
    dout = W.shape[1]

    def body(x_ref, w_ref, o_ref):
        o_ref[...] = jnp.dot(x_ref[...], w_ref[...],
                             preferred_element_type=_f32)

    return pl.pallas_call(
        body,
        grid=(n // _BLK,),
        in_specs=[
            pl.BlockSpec((_BLK, din), lambda i: (i, 0)),
            pl.BlockSpec((din, dout), lambda i: (0, 0)),
        ],
        out_specs=pl.BlockSpec((_BLK, dout), lambda i: (i, 0)),
        out_shape=jax.ShapeDtypeStruct((n, dout), _f32),
    )(x, W)


def _tc_rowscale(y, deg):
    n = y.shape[0]

    def body(y_ref, d_ref, o_ref):
        s = lax.rsqrt(jnp.maximum(d_ref[:, 0:1], 1.0))
        o_ref[...] = y_ref[...] * s

    return pl.pallas_call(
        body,
        grid=(n // _BLK,),
        in_specs=[
            pl.BlockSpec((_BLK, D), lambda i: (i, 0)),
            pl.BlockSpec((_BLK, D), lambda i: (i, 0)),
        ],
        out_specs=pl.BlockSpec((_BLK, D), lambda i: (i, 0)),
        out_shape=jax.ShapeDtypeStruct((n, D), _f32),
    )(y, deg)


def _norm_elu(a_ref, di_ref, b_ref, g_ref, bb_ref):
    si = lax.rsqrt(jnp.maximum(di_ref[:, 0:1], 1.0))
    a = a_ref[...] * si + b_ref[...]
    mu = jnp.mean(a, axis=1, keepdims=True)
    var = jnp.mean((a - mu) ** 2, axis=1, keepdims=True)
    xh = (a - mu) * lax.rsqrt(var + 1e-5) * g_ref[...] + bb_ref[...]
    return jnp.where(xh > 0, xh, jnp.exp(jnp.minimum(xh, 0.0)) - 1.0)


def _tc_combine(agg, indeg16, outdeg16, b, g, bb, Wn):
    n = agg.shape[0]
    blk = 1024

    def body(a_ref, di_ref, do_ref, b_ref, g_ref, bb_ref, w_ref, o_ref):
        h = _norm_elu(a_ref, di_ref, b_ref, g_ref, bb_ref)
        so = lax.rsqrt(jnp.maximum(do_ref[:, 0:1], 1.0))
        o_ref[...] = jnp.dot(h * so, w_ref[...], preferred_element_type=_f32)

    return pl.pallas_call(
        body,
        grid=(n // blk,),
        in_specs=[
            pl.BlockSpec((blk, D), lambda i: (i, 0)),
            pl.BlockSpec((blk, D), lambda i: (i, 0)),
            pl.BlockSpec((blk, D), lambda i: (i, 0)),
            pl.BlockSpec((1, D), lambda i: (0, 0)),
            pl.BlockSpec((1, D), lambda i: (0, 0)),
            pl.BlockSpec((1, D), lambda i: (0, 0)),
            pl.BlockSpec((D, D), lambda i: (0, 0)),
        ],
        out_specs=pl.BlockSpec((blk, D), lambda i: (i, 0)),
        out_shape=jax.ShapeDtypeStruct((n, D), _f32),
    )(agg, indeg16, outdeg16, b, g, bb, Wn)


def _tc_final(aggN, aggL, indegN, indegL, bN, gN, bbN, bL, gL, bbL,
              Wfc, bfc, WoutT, bout):
    n = aggN.shape[0]
    blk = 1024
    steps = n // blk

    def body(an_ref, din_ref, bn_ref, gn_ref, bbn_ref,
             al_ref, dil_ref, bl_ref, gl_ref, bbl_ref,
             wfc_ref, bfc_ref, wout_ref, bout_ref, o_ref, acc_ref):
        i = pl.program_id(0)
        hn = _norm_elu(an_ref, din_ref, bn_ref, gn_ref, bbn_ref)
        hl = _norm_elu(al_ref, dil_ref, bl_ref, gl_ref, bbl_ref)
        row = lax.broadcasted_iota(jnp.int32, (blk, 1), 0) + i * blk
        valid = row < N_NODES
        zero = jnp.zeros_like(hn)
        part = (jnp.sum(jnp.where(valid, hn, zero), axis=0, keepdims=True)
                + jnp.sum(jnp.where(valid, hl, zero), axis=0, keepdims=True))

        @pl.when(i == 0)
        def _():
            acc_ref[0:1, :] = part

        @pl.when(i > 0)
        def _():
            acc_ref[0:1, :] = acc_ref[0:1, :] + part

        @pl.when(i == steps - 1)
        def _():
            hg = acc_ref[0:1, :] * _f32(1.0 / N_NODES)
            z = jnp.maximum(
                jnp.dot(hg, wfc_ref[...], preferred_element_type=_f32)
                + bfc_ref[...], 0.0)
            val = jnp.sum(z * wout_ref[...]) + bout_ref[0, 0]
            o_ref[...] = jnp.full((8, D), val, _f32)

    return pl.pallas_call(
        body,
        grid=(steps,),
        in_specs=[
            pl.BlockSpec((blk, D), lambda i: (i, 0)),
            pl.BlockSpec((blk, D), lambda i: (i, 0)),
            pl.BlockSpec((1, D), lambda i: (0, 0)),
            pl.BlockSpec((1, D), lambda i: (0, 0)),
            pl.BlockSpec((1, D), lambda i: (0, 0)),
            pl.BlockSpec((blk, D), lambda i: (i, 0)),
            pl.BlockSpec((blk, D), lambda i: (i, 0)),
            pl.BlockSpec((1, D), lambda i: (0, 0)),
            pl.BlockSpec((1, D), lambda i: (0, 0)),
            pl.BlockSpec((1, D), lambda i: (0, 0)),
            pl.BlockSpec((D, D), lambda i: (0, 0)),
            pl.BlockSpec((1, D), lambda i: (0, 0)),
            pl.BlockSpec((1, D), lambda i: (0, 0)),
            pl.BlockSpec((1, 1), lambda i: (0, 0)),
        ],
        out_specs=pl.BlockSpec((8, D), lambda i: (0, 0)),
        out_shape=jax.ShapeDtypeStruct((8, D), _f32),
        scratch_shapes=[pltpu.VMEM((8, D), _f32)],
    )(aggN, indegN, bN, gN, bbN, aggL, indegL, bL, gL, bbL,
      Wfc, bfc, WoutT, bout)


def kernel(feat_n, feat_l, edge_index_l2n, edge_index_n2l, params):
    srcA = edge_index_l2n[0].astype(jnp.int32).reshape(NT, SUP, SPC, CHUNK)
    dstA = edge_index_l2n[1].astype(jnp.int32).reshape(NT, SUP, SPC, CHUNK)
    srcB = edge_index_n2l[0].astype(jnp.int32).reshape(NT, SUP, SPC, CHUNK)
    dstB = edge_index_n2l[1].astype(jnp.int32).reshape(NT, SUP, SPC, CHUNK)

    z_h = jnp.zeros((RPT, D), _f32)
    ones_h = jnp.ones((CHUNK, D), _f32)

    degs = _sc_degrees(srcA, dstA, srcB, dstB, ones_h, z_h)
    outdegL, indegN, outdegN, indegL = degs[0], degs[1], degs[2], degs[3]

    def r2(v):
        return v.reshape(1, -1)

    Yl = _tc_rowscale(_tc_matmul(feat_l, params['W_l2n'][0]), outdegL)
    Yn = _tc_rowscale(_tc_matmul(feat_n, params['W_n2l'][0]), outdegN)

    for i in range(2):
        aggN, aggL = _sc_scatter(Yl, Yn, srcA, dstA, srcB, dstB, z_h)
        Yn = _tc_combine(aggN, indegN, outdegN, r2(params['b_l2n'][i]),
                         r2(params['ln_n_g'][i]), r2(params['ln_n_b'][i]),
                         params['W_n2l'][i + 1])
        Yl = _tc_combine(aggL, indegL, outdegL, r2(params['b_n2l'][i]),
                         r2(params['ln_l_g'][i]), r2(params['ln_l_b'][i]),
                         params['W_l2n'][i + 1])

    aggN, aggL = _sc_scatter(Yl, Yn, srcA, dstA, srcB, dstB, z_h)
    out8 = _tc_final(
        aggN, aggL, indegN, indegL,
        r2(params['b_l2n'][2]), r2(params['ln_n_g'][2]), r2(params['ln_n_b'][2]),
        r2(params['b_n2l'][2]), r2(params['ln_l_g'][2]), r2(params['ln_l_b'][2]),
        params['W_fc'], r2(params['b_fc']),
        params['W_out'].reshape(1, D), params['b_out'].reshape(1, 1))
    return out8[0:1, 0:1]
